# Initial kernel scaffold; baseline (speedup 1.0000x reference)
import jax
import jax.numpy as jnp
from jax.experimental import pallas as pl


def kernel(x, edge_index, W1, b1, W2, b2, temp):
    raise NotImplementedError("write your pallas kernel here")


# speedup vs baseline: 9.9210x; 9.9210x over previous
import functools
import math

import jax
import jax.numpy as jnp
import numpy as np
from jax import lax
from jax.experimental import pallas as pl
from jax.experimental.pallas import tpu as pltpu
from jax.experimental.pallas import tpu_sc as plsc

_N = 10000
_E = 320000
_D_IN = 128
_HID = 64
_N_CLS = 40
_K = 10

_NTILE = 16
_NW = 32
_NP = 10112
_R = _NP // _NTILE
_C = 128
_HALF = _HID // 2
_RCH = 128
_NRCH = (_R + _RCH - 1) // _RCH

_EPT = _E // _NTILE
_NSLOT = 3
_NCH = 158
_EPT_PAD = _NCH * _C

_EPW = _E // _NW
_DCH = (_EPW + _C - 1) // _C
_EPW_PAD = _DCH * _C


def _cheb_t_padded():
    xs = np.array([math.cos((_K - j + 0.5) * math.pi / (_K + 1))
                   for j in range(_K + 1)], dtype=np.float64)
    t = np.zeros((_K + 1, _K + 1), dtype=np.float64)
    t[0] = 1.0
    t[1] = xs
    for i in range(2, _K + 1):
        t[i] = 2.0 * xs * t[i - 1] - t[i - 2]
    tp = np.zeros((16, 16), dtype=np.float32)
    tp[:_K + 1, :_K + 1] = t.astype(np.float32)
    return tp


_TMATP = _cheb_t_padded()
_APAT = np.zeros((1, 16), dtype=np.float32)
_APAT[0, 0] = 0.5
for _j in range(2, _K + 1, 2):
    _APAT[0, _j] = -1.0 if (_j // 2) % 2 == 1 else 1.0


def _coe_body(temp_ref, tm_ref, pat_ref, coe_ref, alpha_ref):
    t = jnp.maximum(temp_ref[...], 0.0)
    coe = (2.0 / (_K + 1)) * jnp.dot(t, tm_ref[...])
    coe_ref[...] = coe
    alpha_ref[...] = jnp.sum(coe * pat_ref[...], axis=1, keepdims=True)


def _coe_call(temp16):
    return pl.pallas_call(
        _coe_body,
        out_shape=(
            jax.ShapeDtypeStruct((1, 16), jnp.float32),
            jax.ShapeDtypeStruct((1, 1), jnp.float32),
        ),
    )(temp16, jnp.asarray(_TMATP.T), jnp.asarray(_APAT))


def _front_body(x_ref, w_ref, b_ref, degp_ref, v0_ref, h_ref, rm_ref, dg_ref):
    h = jnp.dot(x_ref[...], w_ref[...],
                preferred_element_type=jnp.float32) + b_ref[...]
    h = jnp.maximum(h, 0.0)
    h_ref[...] = h
    d16 = degp_ref[0, :, :] + degp_ref[1, :, :]
    d = jnp.concatenate([d16, d16], axis=1)
    pos = d > 0.0
    dsafe = jnp.where(pos, d, 1.0)
    s = jnp.where(pos, lax.rsqrt(dsafe), 0.0)
    v0_ref[0, :, :] = s * h[:, :_HALF]
    v0_ref[1, :, :] = s * h[:, _HALF:]
    rm_ref[...] = jnp.where(pos, 1.0 / dsafe, 0.0)
    dg_ref[...] = jnp.concatenate([d, d], axis=1)


def _front_call(xp, W1, b1r, degp):
    blk = _NP // 8
    return pl.pallas_call(
        _front_body,
        grid=(8,),
        in_specs=[
            pl.BlockSpec((blk, _D_IN), lambda i: (i, 0)),
            pl.BlockSpec((_D_IN, _HID), lambda i: (0, 0)),
            pl.BlockSpec((1, _HID), lambda i: (0, 0)),
            pl.BlockSpec((2, blk, 16), lambda i: (0, i, 0)),
        ],
        out_specs=(
            pl.BlockSpec((2, blk, _HALF), lambda i: (0, i, 0)),
            pl.BlockSpec((blk, _HID), lambda i: (i, 0)),
            pl.BlockSpec((blk, _HALF), lambda i: (i, 0)),
            pl.BlockSpec((blk, _HID), lambda i: (i, 0)),
        ),
        out_shape=(
            jax.ShapeDtypeStruct((2, _NP, _HALF), jnp.float32),
            jax.ShapeDtypeStruct((_NP, _HID), jnp.float32),
            jax.ShapeDtypeStruct((_NP, _HALF), jnp.float32),
            jax.ShapeDtypeStruct((_NP, _HID), jnp.float32),
        ),
    )(xp, W1, b1r, degp)


def _final_body(ov_ref, h_ref, dg_ref, al_ref, w_ref, b_ref, o_ref):
    d = dg_ref[...]
    pos = d > 0.0
    hf = jnp.where(pos, jnp.sqrt(jnp.where(pos, d, 1.0)) * ov_ref[...],
                   al_ref[...] * h_ref[...])
    logits = jnp.dot(hf, w_ref[...],
                     preferred_element_type=jnp.float32) + b_ref[...]
    m = jnp.max(logits, axis=1, keepdims=True)
    z = logits - m
    o_ref[...] = z - jnp.log(jnp.sum(jnp.exp(z), axis=1, keepdims=True))


def _final_call(ov, h, dg, alpha, W2, b2r):
    blk = 1000
    return pl.pallas_call(
        _final_body,
        grid=(_N // blk,),
        in_specs=[
            pl.BlockSpec((blk, _HID), lambda i: (i, 0)),
            pl.BlockSpec((blk, _HID), lambda i: (i, 0)),
            pl.BlockSpec((blk, _HID), lambda i: (i, 0)),
            pl.BlockSpec((1, 1), lambda i: (0, 0)),
            pl.BlockSpec((_HID, _N_CLS), lambda i: (0, 0)),
            pl.BlockSpec((1, _N_CLS), lambda i: (0, 0)),
        ],
        out_specs=pl.BlockSpec((blk, _N_CLS), lambda i: (i, 0)),
        out_shape=jax.ShapeDtypeStruct((_N, _N_CLS), jnp.float32),
    )(ov, h, dg, alpha, W2, b2r)


def _deg_body(row_hbm, zer_hbm, dout_hbm, rowv, ones, W):
    cid = lax.axis_index("c")
    sid = lax.axis_index("s")
    wid = cid * _NTILE + sid
    base = sid * _R

    pltpu.sync_copy(row_hbm.at[wid], rowv)
    pltpu.sync_copy(zer_hbm.at[pl.ds(base, _R), pl.ds(0, 16)],
                    W.at[pl.ds(base, _R), :])

    def fill_ones(r, c):
        ones[r, pl.ds(0, 16)] = jnp.full((16,), 1.0, jnp.float32)
        return c
    lax.fori_loop(0, _C, fill_ones, None)

    plsc.subcore_barrier()

    def deg_chunk(j, c):
        pltpu.sync_copy(ones, W.at[rowv.at[j]], add=True)
        return c
    lax.fori_loop(0, _DCH, deg_chunk, None)

    plsc.subcore_barrier()
    pltpu.sync_copy(W.at[pl.ds(base, _R), :],
                    dout_hbm.at[cid, pl.ds(base, _R), :])


_deg_call = functools.partial(
    pl.kernel,
    out_type=jax.ShapeDtypeStruct((2, _NP, 16), jnp.float32),
    mesh=plsc.VectorSubcoreMesh(core_axis_name="c", subcore_axis_name="s"),
    compiler_params=pltpu.CompilerParams(use_tc_tiling_on_sc=False),
    scratch_types=[
        pltpu.VMEM((_DCH, _C), jnp.int32),
        pltpu.VMEM((_C, 16), jnp.float32),
        pltpu.VMEM_SHARED((_NP, 16), jnp.float32),
    ],
)(_deg_body)


def _prop_body(v0_hbm, ec_hbm, coe_hbm, rm_hbm, zer_hbm,
               ov_hbm, zs_hbm,
               Ib, Gb, RM, V0, V1, OT, coev,
               igsem, ggsem, sgsem, W):
    I = [Ib.at[b] for b in range(_NSLOT)]
    G = [Gb.at[b] for b in range(_NSLOT)]
    igs = [igsem.at[b] for b in range(_NSLOT)]
    ggs = [ggsem.at[b] for b in range(_NSLOT)]
    sgs = [sgsem.at[b] for b in range(_NSLOT)]
    gB = Gb.at[2]
    cid = lax.axis_index("c")
    sid = lax.axis_index("s")
    base = sid * _R
    cbase = cid * _NP + base

    pltpu.sync_copy(coe_hbm, coev)
    pltpu.sync_copy(rm_hbm.at[pl.ds(base, _R), :], RM)
    pltpu.sync_copy(v0_hbm.at[cid, pl.ds(base, _R), :], V0)
    pltpu.sync_copy(zer_hbm.at[pl.ds(base, _R), :], W.at[pl.ds(base, _R), :])
    pltpu.sync_copy(V0, zs_hbm.at[pl.ds(cbase, _R), :])

    coe_vec = coev[...]
    c0 = coe_vec[0] * 0.5

    def out0_body(r, c):
        for h in (0, 16):
            OT[r, pl.ds(h, 16)] = c0 * V0[r, pl.ds(h, 16)]
        return c
    lax.fori_loop(0, _R, out0_body, None)

    plsc.subcore_barrier()

    vprev, vcur = V0, V1
    for i in range(1, _K + 1):
        first = (i == 1)
        last = (i == _K)

        def i_issue(j, b):
            pltpu.async_copy(ec_hbm.at[cid, sid, j], I[b], igs[b])

        def i_wait(b):
            pltpu.make_async_copy(ec_hbm.at[cid, sid, 0], I[b],
                                  igs[b]).wait()

        def g_issue(b):
            pltpu.async_copy(zs_hbm.at[Ib.at[b, 0]], G[b], ggs[b])

        def g_wait(b):
            pltpu.make_async_copy(zs_hbm.at[Ib.at[b, 0]], G[b],
                                  ggs[b]).wait()

        def s_issue(b):
            pltpu.async_copy(G[b], W.at[Ib.at[b, 1]], sgs[b], add=True)

        def s_wait(b):
            pltpu.make_async_copy(G[b], W.at[Ib.at[b, 1]], sgs[b]).wait()

        def s_sync(b):
            pltpu.sync_copy(G[b], W.at[Ib.at[b, 1]], add=True)

        i_issue(0, 0)
        i_wait(0)
        i_issue(1, 1)
        g_issue(0)

        def edge_pair(q, c):
            j0 = 2 * q
            g_wait(0)
            i_wait(1)
            g_issue(1)
            s_sync(0)
            i_issue(j0 + 2, 0)
            g_wait(1)
            i_wait(0)
            g_issue(0)
            s_sync(1)
            i_issue(j0 + 3, 1)
            return c
        lax.fori_loop(0, _NCH // 2 - 1, edge_pair, None)
        g_wait(0)
        i_wait(1)
        g_issue(1)
        s_sync(0)
        g_wait(1)
        s_sync(1)

        plsc.subcore_barrier()

        ci = coe_vec[i]
        dst = V1 if first else vprev

        def rb_issue(ch, b):
            rn = min(_RCH, _R - ch * _RCH)
            pltpu.async_copy(W.at[pl.ds(base + ch * _RCH, rn), :],
                             G[b].at[pl.ds(0, rn), :], ggs[b])

        def rb_wait(ch, b):
            rn = min(_RCH, _R - ch * _RCH)
            pltpu.make_async_copy(W.at[pl.ds(base + ch * _RCH, rn), :],
                                  G[b].at[pl.ds(0, rn), :], ggs[b]).wait()

        rb_issue(0, 0)
        for ch in range(_NRCH):
            r0 = ch * _RCH
            rn = min(_RCH, _R - r0)
            b = ch % 2
            rb_wait(ch, b)
            if ch + 1 < _NRCH:
                rb_issue(ch + 1, 1 - b)
            if not last:
                pltpu.sync_copy(zer_hbm.at[pl.ds(base + r0, rn), :],
                                W.at[pl.ds(base + r0, rn), :])

            def upd_body(r, c, r0=r0, b=b, dst=dst, vp=vprev,
                         first=first, ci=ci):
                gb = G[b]
                for h in (0, 16):
                    w = gb[r, pl.ds(h, 16)]
                    rmw = RM[r0 + r, pl.ds(h, 16)] * w
                    if first:
                        tn = -rmw
                    else:
                        tn = (-2.0 * rmw) - vp[r0 + r, pl.ds(h, 16)]
                    dst[r0 + r, pl.ds(h, 16)] = tn
                    OT[r0 + r, pl.ds(h, 16)] = (OT[r0 + r, pl.ds(h, 16)]
                                                + ci * tn)
                return c
            lax.fori_loop(0, rn, upd_body, None)

            if not last:
                pltpu.sync_copy(dst.at[pl.ds(r0, rn), :],
                                zs_hbm.at[pl.ds(cbase + r0, rn), :])

        if first:
            vprev, vcur = V0, V1
        else:
            vprev, vcur = vcur, dst
        plsc.subcore_barrier()

    pltpu.sync_copy(OT, ov_hbm.at[cid, pl.ds(base, _R), :])


_prop_call = functools.partial(
    pl.kernel,
    out_type=(
        jax.ShapeDtypeStruct((2, _NP, _HALF), jnp.float32),
        jax.ShapeDtypeStruct((2 * _NP, _HALF), jnp.float32),
    ),
    mesh=plsc.VectorSubcoreMesh(core_axis_name="c", subcore_axis_name="s"),
    compiler_params=pltpu.CompilerParams(use_tc_tiling_on_sc=False),
    scratch_types=[
        pltpu.VMEM((_NSLOT, 2, _C), jnp.int32),
        pltpu.VMEM((_NSLOT, _C, _HALF), jnp.float32),
        pltpu.VMEM((_R, _HALF), jnp.float32),
        pltpu.VMEM((_R, _HALF), jnp.float32),
        pltpu.VMEM((_R, _HALF), jnp.float32),
        pltpu.VMEM((_R, _HALF), jnp.float32),
        pltpu.VMEM((16,), jnp.float32),
        pltpu.SemaphoreType.DMA((_NSLOT,)),
        pltpu.SemaphoreType.DMA((_NSLOT,)),
        pltpu.SemaphoreType.DMA((_NSLOT,)),
        pltpu.VMEM_SHARED((_NP, _HALF), jnp.float32),
    ],
)(_prop_body)


def _pad_edges(idx, nseg, npad):
    seg = idx.reshape(nseg, _E // nseg)
    pad = jnp.full((nseg, npad - _E // nseg), _N, jnp.int32)
    return jnp.concatenate([seg, pad], axis=1).reshape(nseg, npad // _C, _C)


def _edge_chunks(edge_index):
    row = _pad_edges(edge_index[0], _NTILE, _EPT_PAD)
    col = _pad_edges(edge_index[1], _NTILE, _EPT_PAD)
    rc0 = jnp.stack([row, col], axis=2)
    rc1 = jnp.stack([row + _NP, col], axis=2)
    return jnp.stack([rc0, rc1], axis=0)


def kernel(x, edge_index, W1, b1, W2, b2, temp):
    xp = jnp.pad(x, ((0, _NP - _N), (0, 0)))
    zeros = jnp.zeros((_NP, _HALF), jnp.float32)

    rowd = _pad_edges(edge_index[0], _NW, _EPW_PAD)
    degp = _deg_call(rowd, zeros)

    v0, h64, rm, dg = _front_call(xp, W1, b1.reshape(1, _HID), degp)
    coe16, alpha = _coe_call(jnp.pad(temp, (0, 16 - (_K + 1))).reshape(1, 16))

    ov, _zs = _prop_call(v0, _edge_chunks(edge_index), coe16.reshape(16),
                         rm, zeros)

    ovc = jnp.concatenate([ov[0, :_N], ov[1, :_N]], axis=1)
    return _final_call(ovc, h64[:_N], dg[:_N], alpha,
                       W2, b2.reshape(1, _N_CLS))
